# baseline (speedup 1.0000x reference)
import functools

import jax
import jax.numpy as jnp
from jax import lax
from jax.experimental import pallas as pl
from jax.experimental.pallas import tpu as pltpu
from jax.experimental.pallas import tpu_sc as plsc

HID = 128
N_NODES = 10000
N_EDGES = 320000
NPAD = 10240
ET = 512
NT = 512

_NC = 2
_NS = 16
_NW = _NC * _NS
_EW = N_EDGES // _NW
_CH = 80
_GB = 5
_BLK = _CH * _GB
_NB = _EW // _BLK
_GBS = 2
_BLKS = _CH * _GBS
_NBS = _EW // _BLKS
_SLAB = NPAD // _NS

_EPS = 1e-5


def _full(shape):
    return pl.BlockSpec(shape, lambda i: (0, 0))


def _mlp_ln_body(x_ref, w1_ref, b1_ref, w2_ref, b2_ref, g_ref, be_ref, o_ref):
    x = x_ref[...]
    h = jnp.maximum(
        jnp.dot(x, w1_ref[...], preferred_element_type=jnp.float32) + b1_ref[...], 0.0)
    h2 = jnp.dot(h, w2_ref[...], preferred_element_type=jnp.float32) + b2_ref[...]
    mu = jnp.mean(h2, axis=1, keepdims=True)
    var = jnp.mean((h2 - mu) * (h2 - mu), axis=1, keepdims=True)
    o_ref[...] = (h2 - mu) * lax.rsqrt(var + _EPS) * g_ref[...] + be_ref[...]


def _encode(x, w1, b1, w2, b2, g, be, tile):
    n, k = x.shape
    return pl.pallas_call(
        _mlp_ln_body,
        grid=(n // tile,),
        in_specs=[
            pl.BlockSpec((tile, k), lambda i: (i, 0)),
            _full(w1.shape), _full((1, HID)), _full((HID, HID)),
            _full((1, HID)), _full((1, HID)), _full((1, HID)),
        ],
        out_specs=pl.BlockSpec((tile, HID), lambda i: (i, 0)),
        out_shape=jax.ShapeDtypeStruct((n, HID), jnp.float32),
    )(x, w1, b1.reshape(1, -1), w2, b2.reshape(1, -1),
      g.reshape(1, -1), be.reshape(1, -1))


def _proj_body(v_ref, ws_ref, wd_ref, os_ref, od_ref):
    v = v_ref[...]
    os_ref[...] = jnp.dot(v, ws_ref[...], preferred_element_type=jnp.float32)
    od_ref[...] = jnp.dot(v, wd_ref[...], preferred_element_type=jnp.float32)


def _proj(v, ws, wd):
    sds = jax.ShapeDtypeStruct((NPAD, HID), jnp.float32)
    return pl.pallas_call(
        _proj_body,
        grid=(NPAD // NT,),
        in_specs=[
            pl.BlockSpec((NT, HID), lambda i: (i, 0)),
            _full((HID, HID)), _full((HID, HID)),
        ],
        out_specs=[pl.BlockSpec((NT, HID), lambda i: (i, 0))] * 2,
        out_shape=[sds, sds],
    )(v, ws, wd)


def _edge_body(e_ref, g_ref, w1_ref, b1_ref, w2_ref, b2_ref, ga_ref,
               be_ref, o_ref):
    x = e_ref[...]
    t = jnp.dot(x.astype(jnp.bfloat16), w1_ref[...].astype(jnp.bfloat16),
                preferred_element_type=jnp.float32) \
        + g_ref[...] + b1_ref[...]
    h = jnp.maximum(t, 0.0)
    h2 = jnp.dot(h.astype(jnp.bfloat16), w2_ref[...].astype(jnp.bfloat16),
                 preferred_element_type=jnp.float32) + b2_ref[...]
    mu = jnp.mean(h2, axis=1, keepdims=True)
    var = jnp.mean((h2 - mu) * (h2 - mu), axis=1, keepdims=True)
    o_ref[...] = x + (h2 - mu) * lax.rsqrt(var + _EPS) * ga_ref[...] + be_ref[...]


def _edge_stage(e, g, w1e, b1, w2, b2, ga, be):
    return pl.pallas_call(
        _edge_body,
        grid=(N_EDGES // ET,),
        in_specs=[
            pl.BlockSpec((ET, HID), lambda i: (i, 0)),
            pl.BlockSpec((ET, HID), lambda i: (i, 0)),
            _full((HID, HID)), _full((1, HID)), _full((HID, HID)),
            _full((1, HID)), _full((1, HID)), _full((1, HID)),
        ],
        out_specs=pl.BlockSpec((ET, HID), lambda i: (i, 0)),
        out_shape=jax.ShapeDtypeStruct((N_EDGES, HID), jnp.float32),
    )(e, g, w1e, b1.reshape(1, -1), w2, b2.reshape(1, -1),
      ga.reshape(1, -1), be.reshape(1, -1))


def _node_body(v_ref, p_ref, wv_ref, wa_ref, b1_ref, w2_ref, b2_ref, ga_ref,
               be_ref, o_ref):
    v = v_ref[...]
    agg = p_ref[0] + p_ref[1]
    t = jnp.dot(v, wv_ref[...], preferred_element_type=jnp.float32) \
        + jnp.dot(agg, wa_ref[...], preferred_element_type=jnp.float32) \
        + b1_ref[...]
    h = jnp.maximum(t, 0.0)
    h2 = jnp.dot(h, w2_ref[...], preferred_element_type=jnp.float32) + b2_ref[...]
    mu = jnp.mean(h2, axis=1, keepdims=True)
    var = jnp.mean((h2 - mu) * (h2 - mu), axis=1, keepdims=True)
    o_ref[...] = v + (h2 - mu) * lax.rsqrt(var + _EPS) * ga_ref[...] + be_ref[...]


def _node_stage(v, p, wv, wa, b1, w2, b2, ga, be):
    return pl.pallas_call(
        _node_body,
        grid=(NPAD // NT,),
        in_specs=[
            pl.BlockSpec((NT, HID), lambda i: (i, 0)),
            pl.BlockSpec((2, NT, HID), lambda i: (0, i, 0)),
            _full((HID, HID)), _full((HID, HID)), _full((1, HID)),
            _full((HID, HID)), _full((1, HID)), _full((1, HID)), _full((1, HID)),
        ],
        out_specs=pl.BlockSpec((NT, HID), lambda i: (i, 0)),
        out_shape=jax.ShapeDtypeStruct((NPAD, HID), jnp.float32),
    )(v, p, wv, wa, b1.reshape(1, -1), w2, b2.reshape(1, -1),
      ga.reshape(1, -1), be.reshape(1, -1))


def _node_proj_body(v_ref, p_ref, wv_ref, wa_ref, b1_ref, w2_ref, b2_ref,
                    ga_ref, be_ref, ws_ref, wd_ref, o_ref, os_ref, od_ref):
    v = v_ref[...]
    agg = p_ref[0] + p_ref[1]
    t = jnp.dot(v, wv_ref[...], preferred_element_type=jnp.float32) \
        + jnp.dot(agg, wa_ref[...], preferred_element_type=jnp.float32) \
        + b1_ref[...]
    h = jnp.maximum(t, 0.0)
    h2 = jnp.dot(h, w2_ref[...], preferred_element_type=jnp.float32) + b2_ref[...]
    mu = jnp.mean(h2, axis=1, keepdims=True)
    var = jnp.mean((h2 - mu) * (h2 - mu), axis=1, keepdims=True)
    vn = v + (h2 - mu) * lax.rsqrt(var + _EPS) * ga_ref[...] + be_ref[...]
    o_ref[...] = vn
    os_ref[...] = jnp.dot(vn, ws_ref[...], preferred_element_type=jnp.float32)
    od_ref[...] = jnp.dot(vn, wd_ref[...], preferred_element_type=jnp.float32)


def _node_proj_stage(v, p, wv, wa, b1, w2, b2, ga, be, ws, wd):
    tspec = pl.BlockSpec((NT, HID), lambda i: (i, 0))
    hspec = pl.BlockSpec((NT, HID), lambda i: (i, 0))
    return pl.pallas_call(
        _node_proj_body,
        grid=(NPAD // NT,),
        in_specs=[
            tspec,
            pl.BlockSpec((2, NT, HID), lambda i: (0, i, 0)),
            _full((HID, HID)), _full((HID, HID)), _full((1, HID)),
            _full((HID, HID)), _full((1, HID)), _full((1, HID)), _full((1, HID)),
            _full((HID, HID)), _full((HID, HID)),
        ],
        out_specs=[tspec, hspec, hspec],
        out_shape=[
            jax.ShapeDtypeStruct((NPAD, HID), jnp.float32),
            jax.ShapeDtypeStruct((NPAD, HID), jnp.float32),
            jax.ShapeDtypeStruct((NPAD, HID), jnp.float32),
        ],
    )(v, p, wv, wa, b1.reshape(1, -1), w2, b2.reshape(1, -1),
      ga.reshape(1, -1), be.reshape(1, -1), ws, wd)


def _dec_body(v_ref, w1_ref, b1_ref, w2_ref, b2_ref, o_ref):
    h = jnp.maximum(
        jnp.dot(v_ref[...], w1_ref[...], preferred_element_type=jnp.float32)
        + b1_ref[...], 0.0)
    o_ref[...] = jnp.dot(h, w2_ref[...], preferred_element_type=jnp.float32) \
        + b2_ref[...]


def _decode(v, w1, b1, w2p, b2p):
    return pl.pallas_call(
        _dec_body,
        grid=(NPAD // NT,),
        in_specs=[
            pl.BlockSpec((NT, HID), lambda i: (i, 0)),
            _full((HID, HID)), _full((1, HID)), _full((HID, HID)), _full((1, HID)),
        ],
        out_specs=pl.BlockSpec((NT, HID), lambda i: (i, 0)),
        out_shape=jax.ShapeDtypeStruct((NPAD, HID), jnp.float32),
    )(v, w1, b1.reshape(1, -1), w2p, b2p.reshape(1, -1))


def _sc_gather_body(ms_hbm, md_hbm, src_hbm, dst_hbm, out_hbm,
                    ia0, ia1, ib0, ib1, r0, r1, tbl,
                    sem_i, sem_g, sem_h, sem_s):
    sid = lax.axis_index("s")
    w = lax.axis_index("c") * _NS + sid
    ia = (ia0, ia1)
    ib = (ib0, ib1)
    rw = (r0, r1)
    pltpu.sync_copy(ms_hbm.at[pl.ds(sid * _SLAB, _SLAB)],
                    tbl.at[pl.ds(sid * _SLAB, _SLAB)])
    plsc.subcore_barrier()

    def fire_idx(kb, p):
        base = w * _EW + kb * _BLKS
        pltpu.async_copy(src_hbm.at[pl.ds(base, _BLKS)], ia[p], sem_i)
        pltpu.async_copy(dst_hbm.at[pl.ds(base, _BLKS)], ib[p], sem_i)

    def drain_idx(p):
        pltpu.make_async_copy(src_hbm.at[pl.ds(0, _BLKS)], ia[p], sem_i).wait()
        pltpu.make_async_copy(dst_hbm.at[pl.ds(0, _BLKS)], ib[p], sem_i).wait()

    def fire_sg(p):
        for j in range(_GBS):
            pltpu.async_copy(tbl.at[ia[p].at[pl.ds(j * _CH, _CH)]],
                             rw[p].at[pl.ds(j * _CH, _CH)], sem_g)

    def drain_sg(p):
        for j in range(_GBS):
            pltpu.make_async_copy(tbl.at[ia[p].at[pl.ds(j * _CH, _CH)]],
                                  rw[p].at[pl.ds(j * _CH, _CH)], sem_g).wait()

    def fire_ha(p):
        for j in range(_GBS):
            pltpu.async_copy(md_hbm.at[ib[p].at[pl.ds(j * _CH, _CH)]],
                             rw[p].at[pl.ds(j * _CH, _CH)], sem_h, add=True)

    def drain_ha(p):
        for j in range(_GBS):
            pltpu.make_async_copy(md_hbm.at[ib[p].at[pl.ds(j * _CH, _CH)]],
                                  rw[p].at[pl.ds(j * _CH, _CH)], sem_h).wait()

    def fire_store(kb, p):
        base = w * _EW + kb * _BLKS
        pltpu.async_copy(rw[p], out_hbm.at[pl.ds(base, _BLKS)], sem_s)

    def drain_store(p):
        pltpu.make_async_copy(rw[p], out_hbm.at[pl.ds(0, _BLKS)], sem_s).wait()

    fire_idx(0, 0)
    drain_idx(0)
    fire_sg(0)

    def body(i, carry):
        k0 = 2 * i
        k1 = k0 + 1
        fire_idx(k1, 1)
        drain_sg(0)
        fire_ha(0)
        drain_idx(1)
        fire_sg(1)
        drain_ha(0)
        fire_idx(k0 + 2, 0)
        fire_store(k0, 0)
        drain_sg(1)
        fire_ha(1)
        drain_idx(0)
        drain_store(0)
        fire_sg(0)
        drain_ha(1)
        fire_store(k1, 1)
        drain_store(1)
        return carry

    lax.fori_loop(0, (_NBS - 2) // 2, body, 0)
    fire_idx(_NBS - 1, 1)
    drain_sg(0)
    fire_ha(0)
    drain_idx(1)
    fire_sg(1)
    drain_ha(0)
    fire_store(_NBS - 2, 0)
    drain_sg(1)
    fire_ha(1)
    drain_ha(1)
    drain_store(0)
    fire_store(_NBS - 1, 1)
    tbase = w * _EW + _NBS * _BLKS
    pltpu.sync_copy(src_hbm.at[pl.ds(tbase, _CH)], ia[0].at[pl.ds(0, _CH)])
    pltpu.sync_copy(dst_hbm.at[pl.ds(tbase, _CH)], ib[0].at[pl.ds(0, _CH)])
    pltpu.async_copy(tbl.at[ia[0].at[pl.ds(0, _CH)]],
                     r0.at[pl.ds(0, _CH)], sem_g).wait()
    pltpu.async_copy(md_hbm.at[ib[0].at[pl.ds(0, _CH)]],
                     r0.at[pl.ds(0, _CH)], sem_h, add=True).wait()
    pltpu.sync_copy(r0.at[pl.ds(0, _CH)], out_hbm.at[pl.ds(tbase, _CH)])
    drain_store(1)


def _sc_scatter_body(e_hbm, dst_hbm, zer_hbm, p_hbm, ix0, ix1, r0, r1, acc,
                     sem_i, sem_r, sem_c):
    cid = lax.axis_index("c")
    sid = lax.axis_index("s")
    w = cid * _NS + sid
    ix = (ix0, ix1)
    rw = (r0, r1)
    pltpu.sync_copy(zer_hbm.at[pl.ds(sid * _SLAB, _SLAB)],
                    acc.at[pl.ds(sid * _SLAB, _SLAB)])
    plsc.subcore_barrier()

    def fire_loads(kb, p):
        base = w * _EW + kb * _BLKS
        for j in range(_GBS):
            pltpu.async_copy(dst_hbm.at[pl.ds(base + j * _CH, _CH)],
                             ix[p][j], sem_i)
        pltpu.async_copy(e_hbm.at[pl.ds(base, _BLKS)], rw[p], sem_r)

    def drain_loads(p):
        for j in range(_GBS):
            pltpu.make_async_copy(dst_hbm.at[pl.ds(0, _CH)], ix[p][j],
                                  sem_i).wait()
        pltpu.make_async_copy(e_hbm.at[pl.ds(0, _BLKS)], rw[p], sem_r).wait()

    def fire_sadd(p):
        for j in range(_GBS):
            pltpu.async_copy(rw[p].at[pl.ds(j * _CH, _CH)],
                             acc.at[ix[p][j]], sem_c, add=True)

    def drain_sadd(p):
        for j in range(_GBS):
            pltpu.make_async_copy(rw[p].at[pl.ds(j * _CH, _CH)],
                                  acc.at[ix[p][j]], sem_c).wait()

    fire_loads(0, 0)
    drain_loads(0)

    def body(i, carry):
        k1 = 2 * i + 1
        fire_loads(k1, 1)
        fire_sadd(0)
        drain_loads(1)
        drain_sadd(0)
        fire_loads(k1 + 1, 0)
        fire_sadd(1)
        drain_loads(0)
        drain_sadd(1)
        return carry

    lax.fori_loop(0, (_NBS - 2) // 2, body, 0)
    fire_loads(_NBS - 1, 1)
    fire_sadd(0)
    drain_loads(1)
    drain_sadd(0)
    fire_sadd(1)
    drain_sadd(1)
    tbase = w * _EW + _NBS * _BLKS
    pltpu.sync_copy(dst_hbm.at[pl.ds(tbase, _CH)], ix[0][0])
    pltpu.sync_copy(e_hbm.at[pl.ds(tbase, _CH)], r0.at[pl.ds(0, _CH)])
    pltpu.async_copy(r0.at[pl.ds(0, _CH)], acc.at[ix[0][0]], sem_c,
                     add=True).wait()
    plsc.subcore_barrier()
    pltpu.sync_copy(acc.at[pl.ds(sid * _SLAB, _SLAB)],
                    p_hbm.at[cid, pl.ds(sid * _SLAB, _SLAB)])


@functools.cache
def _sc_kernels():
    mesh = plsc.VectorSubcoreMesh(core_axis_name="c", subcore_axis_name="s",
                                  num_cores=_NC, num_subcores=_NS)
    gather = pl.kernel(
        _sc_gather_body,
        out_type=jax.ShapeDtypeStruct((N_EDGES, HID), jnp.float32),
        mesh=mesh,
        scratch_types=[
            pltpu.VMEM((_BLKS,), jnp.int32),
            pltpu.VMEM((_BLKS,), jnp.int32),
            pltpu.VMEM((_BLKS,), jnp.int32),
            pltpu.VMEM((_BLKS,), jnp.int32),
            pltpu.VMEM((_BLKS, HID), jnp.float32),
            pltpu.VMEM((_BLKS, HID), jnp.float32),
            pltpu.VMEM_SHARED((NPAD, HID), jnp.float32),
            pltpu.SemaphoreType.DMA,
            pltpu.SemaphoreType.DMA,
            pltpu.SemaphoreType.DMA,
            pltpu.SemaphoreType.DMA,
        ],
    )
    scatter = pl.kernel(
        _sc_scatter_body,
        out_type=jax.ShapeDtypeStruct((_NC, NPAD, HID), jnp.float32),
        mesh=mesh,
        scratch_types=[
            [pltpu.VMEM((_CH,), jnp.int32) for _ in range(_GBS)],
            [pltpu.VMEM((_CH,), jnp.int32) for _ in range(_GBS)],
            pltpu.VMEM((_BLKS, HID), jnp.float32),
            pltpu.VMEM((_BLKS, HID), jnp.float32),
            pltpu.VMEM_SHARED((NPAD, HID), jnp.float32),
            pltpu.SemaphoreType.DMA,
            pltpu.SemaphoreType.DMA,
            pltpu.SemaphoreType.DMA,
        ],
    )
    return gather, scatter


def kernel(node_attr, edge_attr, edge_index, params):
    sc_gather, sc_scatter = _sc_kernels()
    src = edge_index[0]
    dst = edge_index[1]

    na = jnp.pad(node_attr, ((0, NPAD - N_NODES), (0, 5)))
    ea = jnp.pad(edge_attr, ((0, 0), (0, 1)))

    pn = params['node_enc']
    v = _encode(na, jnp.pad(pn['w1'], ((0, 5), (0, 0))), pn['b1'], pn['w2'],
                pn['b2'], pn['g'], pn['beta'], NT)
    pe = params['edge_enc']
    e = _encode(ea, jnp.pad(pe['w1'], ((0, 1), (0, 0))), pe['b1'], pe['w2'],
                pe['b2'], pe['g'], pe['beta'], ET)

    zer = jnp.zeros((NPAD, HID), jnp.float32)
    blocks = params['blocks']
    w10 = blocks[0]['edge']['w1']
    ms, md = _proj(v, w10[HID:2 * HID], w10[2 * HID:])
    for bi, blk in enumerate(blocks):
        nxt = blocks[bi + 1] if bi + 1 < len(blocks) else None
        be_, bn = blk['edge'], blk['node']
        w1 = be_['w1']
        g = sc_gather(ms, md, src, dst)
        e = _edge_stage(e, g, w1[:HID], be_['b1'], be_['w2'], be_['b2'],
                        be_['g'], be_['beta'])
        p = sc_scatter(e, dst, zer)
        nw1 = bn['w1']
        if nxt is not None:
            nw1e = nxt['edge']['w1']
            v, ms, md = _node_proj_stage(
                v, p, nw1[:HID], nw1[HID:], bn['b1'], bn['w2'], bn['b2'],
                bn['g'], bn['beta'], nw1e[HID:2 * HID], nw1e[2 * HID:])
        else:
            v = _node_stage(v, p, nw1[:HID], nw1[HID:], bn['b1'], bn['w2'],
                            bn['b2'], bn['g'], bn['beta'])

    pd = params['decoder']
    w2p = jnp.pad(pd['w2'], ((0, 0), (0, HID - pd['w2'].shape[1])))
    b2p = jnp.pad(pd['b2'], (0, HID - pd['b2'].shape[0]))
    out = _decode(v, pd['w1'], pd['b1'], w2p, b2p)
    return out[:N_NODES, :3]


# speedup vs baseline: 1.1123x; 1.1123x over previous
import functools

import jax
import jax.numpy as jnp
from jax import lax
from jax.experimental import pallas as pl
from jax.experimental.pallas import tpu as pltpu
from jax.experimental.pallas import tpu_sc as plsc

HID = 128
N_NODES = 10000
N_EDGES = 320000
NPAD = 10240
ET = 512
NT = 512

_NC = 2
_NS = 16
_NW = _NC * _NS
_EW = N_EDGES // _NW
_CH = 80
_GB = 5
_BLK = _CH * _GB
_NB = _EW // _BLK
_GBS = 2
_BLKS = _CH * _GBS
_NBS = _EW // _BLKS
_SLAB = NPAD // _NS

_EPS = 1e-5


def _full(shape):
    return pl.BlockSpec(shape, lambda i: (0, 0))


def _mlp_ln_body(x_ref, w1_ref, b1_ref, w2_ref, b2_ref, g_ref, be_ref, o_ref):
    x = x_ref[...]
    h = jnp.maximum(
        jnp.dot(x, w1_ref[...], preferred_element_type=jnp.float32) + b1_ref[...], 0.0)
    h2 = jnp.dot(h, w2_ref[...], preferred_element_type=jnp.float32) + b2_ref[...]
    mu = jnp.mean(h2, axis=1, keepdims=True)
    var = jnp.mean((h2 - mu) * (h2 - mu), axis=1, keepdims=True)
    o_ref[...] = (h2 - mu) * lax.rsqrt(var + _EPS) * g_ref[...] + be_ref[...]


def _encode(x, w1, b1, w2, b2, g, be, tile):
    n, k = x.shape
    return pl.pallas_call(
        _mlp_ln_body,
        grid=(n // tile,),
        in_specs=[
            pl.BlockSpec((tile, k), lambda i: (i, 0)),
            _full(w1.shape), _full((1, HID)), _full((HID, HID)),
            _full((1, HID)), _full((1, HID)), _full((1, HID)),
        ],
        out_specs=pl.BlockSpec((tile, HID), lambda i: (i, 0)),
        out_shape=jax.ShapeDtypeStruct((n, HID), jnp.float32),
    )(x, w1, b1.reshape(1, -1), w2, b2.reshape(1, -1),
      g.reshape(1, -1), be.reshape(1, -1))


def _proj_body(v_ref, ws_ref, wd_ref, os_ref, od_ref):
    v = v_ref[...]
    os_ref[...] = jnp.dot(v, ws_ref[...], preferred_element_type=jnp.float32)
    od_ref[...] = jnp.dot(v, wd_ref[...], preferred_element_type=jnp.float32)


def _proj(v, ws, wd):
    sds = jax.ShapeDtypeStruct((NPAD, HID), jnp.float32)
    return pl.pallas_call(
        _proj_body,
        grid=(NPAD // NT,),
        in_specs=[
            pl.BlockSpec((NT, HID), lambda i: (i, 0)),
            _full((HID, HID)), _full((HID, HID)),
        ],
        out_specs=[pl.BlockSpec((NT, HID), lambda i: (i, 0))] * 2,
        out_shape=[sds, sds],
    )(v, ws, wd)


def _edge_body(e_ref, g_ref, w1_ref, b1_ref, w2_ref, b2_ref, ga_ref,
               be_ref, o_ref):
    x = e_ref[...]
    t = jnp.dot(x, w1_ref[...], preferred_element_type=jnp.float32) \
        + g_ref[...] + b1_ref[...]
    h = jnp.maximum(t, 0.0)
    h2 = jnp.dot(h, w2_ref[...], preferred_element_type=jnp.float32) + b2_ref[...]
    mu = jnp.mean(h2, axis=1, keepdims=True)
    var = jnp.mean((h2 - mu) * (h2 - mu), axis=1, keepdims=True)
    o_ref[...] = x + (h2 - mu) * lax.rsqrt(var + _EPS) * ga_ref[...] + be_ref[...]


def _edge_stage(e, g, w1e, b1, w2, b2, ga, be):
    n = e.shape[0]
    return pl.pallas_call(
        _edge_body,
        grid=(n // ET,),
        in_specs=[
            pl.BlockSpec((ET, HID), lambda i: (i, 0)),
            pl.BlockSpec((ET, HID), lambda i: (i, 0)),
            _full((HID, HID)), _full((1, HID)), _full((HID, HID)),
            _full((1, HID)), _full((1, HID)), _full((1, HID)),
        ],
        out_specs=pl.BlockSpec((ET, HID), lambda i: (i, 0)),
        out_shape=jax.ShapeDtypeStruct((n, HID), jnp.float32),
    )(e, g, w1e, b1.reshape(1, -1), w2, b2.reshape(1, -1),
      ga.reshape(1, -1), be.reshape(1, -1))


def _node_body(v_ref, pa_ref, pb_ref, wv_ref, wa_ref, b1_ref, w2_ref, b2_ref,
               ga_ref, be_ref, o_ref):
    v = v_ref[...]
    agg = pa_ref[0] + pa_ref[1] + pb_ref[0] + pb_ref[1]
    t = jnp.dot(v, wv_ref[...], preferred_element_type=jnp.float32) \
        + jnp.dot(agg, wa_ref[...], preferred_element_type=jnp.float32) \
        + b1_ref[...]
    h = jnp.maximum(t, 0.0)
    h2 = jnp.dot(h, w2_ref[...], preferred_element_type=jnp.float32) + b2_ref[...]
    mu = jnp.mean(h2, axis=1, keepdims=True)
    var = jnp.mean((h2 - mu) * (h2 - mu), axis=1, keepdims=True)
    o_ref[...] = v + (h2 - mu) * lax.rsqrt(var + _EPS) * ga_ref[...] + be_ref[...]


def _node_stage(v, pa, pb, wv, wa, b1, w2, b2, ga, be):
    return pl.pallas_call(
        _node_body,
        grid=(NPAD // NT,),
        in_specs=[
            pl.BlockSpec((NT, HID), lambda i: (i, 0)),
            pl.BlockSpec((2, NT, HID), lambda i: (0, i, 0)),
            pl.BlockSpec((2, NT, HID), lambda i: (0, i, 0)),
            _full((HID, HID)), _full((HID, HID)), _full((1, HID)),
            _full((HID, HID)), _full((1, HID)), _full((1, HID)), _full((1, HID)),
        ],
        out_specs=pl.BlockSpec((NT, HID), lambda i: (i, 0)),
        out_shape=jax.ShapeDtypeStruct((NPAD, HID), jnp.float32),
    )(v, pa, pb, wv, wa, b1.reshape(1, -1), w2, b2.reshape(1, -1),
      ga.reshape(1, -1), be.reshape(1, -1))


def _node_proj_body(v_ref, pa_ref, pb_ref, wv_ref, wa_ref, b1_ref, w2_ref,
                    b2_ref, ga_ref, be_ref, ws_ref, wd_ref, o_ref, os_ref,
                    od_ref):
    v = v_ref[...]
    agg = pa_ref[0] + pa_ref[1] + pb_ref[0] + pb_ref[1]
    t = jnp.dot(v, wv_ref[...], preferred_element_type=jnp.float32) \
        + jnp.dot(agg, wa_ref[...], preferred_element_type=jnp.float32) \
        + b1_ref[...]
    h = jnp.maximum(t, 0.0)
    h2 = jnp.dot(h, w2_ref[...], preferred_element_type=jnp.float32) + b2_ref[...]
    mu = jnp.mean(h2, axis=1, keepdims=True)
    var = jnp.mean((h2 - mu) * (h2 - mu), axis=1, keepdims=True)
    vn = v + (h2 - mu) * lax.rsqrt(var + _EPS) * ga_ref[...] + be_ref[...]
    o_ref[...] = vn
    os_ref[...] = jnp.dot(vn, ws_ref[...], preferred_element_type=jnp.float32)
    od_ref[...] = jnp.dot(vn, wd_ref[...], preferred_element_type=jnp.float32)


def _node_proj_stage(v, pa, pb, wv, wa, b1, w2, b2, ga, be, ws, wd):
    tspec = pl.BlockSpec((NT, HID), lambda i: (i, 0))
    hspec = pl.BlockSpec((NT, HID), lambda i: (i, 0))
    return pl.pallas_call(
        _node_proj_body,
        grid=(NPAD // NT,),
        in_specs=[
            tspec,
            pl.BlockSpec((2, NT, HID), lambda i: (0, i, 0)),
            pl.BlockSpec((2, NT, HID), lambda i: (0, i, 0)),
            _full((HID, HID)), _full((HID, HID)), _full((1, HID)),
            _full((HID, HID)), _full((1, HID)), _full((1, HID)), _full((1, HID)),
            _full((HID, HID)), _full((HID, HID)),
        ],
        out_specs=[tspec, hspec, hspec],
        out_shape=[
            jax.ShapeDtypeStruct((NPAD, HID), jnp.float32),
            jax.ShapeDtypeStruct((NPAD, HID), jnp.float32),
            jax.ShapeDtypeStruct((NPAD, HID), jnp.float32),
        ],
    )(v, pa, pb, wv, wa, b1.reshape(1, -1), w2, b2.reshape(1, -1),
      ga.reshape(1, -1), be.reshape(1, -1), ws, wd)


def _dec_body(v_ref, w1_ref, b1_ref, w2_ref, b2_ref, o_ref):
    h = jnp.maximum(
        jnp.dot(v_ref[...], w1_ref[...], preferred_element_type=jnp.float32)
        + b1_ref[...], 0.0)
    o_ref[...] = jnp.dot(h, w2_ref[...], preferred_element_type=jnp.float32) \
        + b2_ref[...]


def _decode(v, w1, b1, w2p, b2p):
    return pl.pallas_call(
        _dec_body,
        grid=(NPAD // NT,),
        in_specs=[
            pl.BlockSpec((NT, HID), lambda i: (i, 0)),
            _full((HID, HID)), _full((1, HID)), _full((HID, HID)), _full((1, HID)),
        ],
        out_specs=pl.BlockSpec((NT, HID), lambda i: (i, 0)),
        out_shape=jax.ShapeDtypeStruct((NPAD, HID), jnp.float32),
    )(v, w1, b1.reshape(1, -1), w2p, b2p.reshape(1, -1))


def _sc_bodies(ew, nb, tail):
    def gather_body(ms_hbm, md_hbm, src_hbm, dst_hbm, out_hbm,
                    ia0, ia1, ib0, ib1, r0, r1, tbl,
                    sem_i, sem_g, sem_h, sem_s):
        sid = lax.axis_index("s")
        w = lax.axis_index("c") * _NS + sid
        ia = (ia0, ia1)
        ib = (ib0, ib1)
        rw = (r0, r1)
        pltpu.sync_copy(ms_hbm.at[pl.ds(sid * _SLAB, _SLAB)],
                        tbl.at[pl.ds(sid * _SLAB, _SLAB)])
        plsc.subcore_barrier()

        def fire_idx(kb, p):
            base = w * ew + kb * _BLKS
            pltpu.async_copy(src_hbm.at[pl.ds(base, _BLKS)], ia[p], sem_i)
            pltpu.async_copy(dst_hbm.at[pl.ds(base, _BLKS)], ib[p], sem_i)

        def drain_idx(p):
            pltpu.make_async_copy(src_hbm.at[pl.ds(0, _BLKS)], ia[p],
                                  sem_i).wait()
            pltpu.make_async_copy(dst_hbm.at[pl.ds(0, _BLKS)], ib[p],
                                  sem_i).wait()

        def fire_sg(p):
            for j in range(_GBS):
                pltpu.async_copy(tbl.at[ia[p].at[pl.ds(j * _CH, _CH)]],
                                 rw[p].at[pl.ds(j * _CH, _CH)], sem_g)

        def drain_sg(p):
            for j in range(_GBS):
                pltpu.make_async_copy(tbl.at[ia[p].at[pl.ds(j * _CH, _CH)]],
                                      rw[p].at[pl.ds(j * _CH, _CH)],
                                      sem_g).wait()

        def fire_ha(p):
            for j in range(_GBS):
                pltpu.async_copy(md_hbm.at[ib[p].at[pl.ds(j * _CH, _CH)]],
                                 rw[p].at[pl.ds(j * _CH, _CH)], sem_h,
                                 add=True)

        def drain_ha(p):
            for j in range(_GBS):
                pltpu.make_async_copy(md_hbm.at[ib[p].at[pl.ds(j * _CH, _CH)]],
                                      rw[p].at[pl.ds(j * _CH, _CH)],
                                      sem_h).wait()

        def fire_store(kb, p):
            base = w * ew + kb * _BLKS
            pltpu.async_copy(rw[p], out_hbm.at[pl.ds(base, _BLKS)], sem_s)

        def drain_store(p):
            pltpu.make_async_copy(rw[p], out_hbm.at[pl.ds(0, _BLKS)],
                                  sem_s).wait()

        fire_idx(0, 0)
        drain_idx(0)
        fire_sg(0)

        def body(i, carry):
            k0 = 2 * i
            k1 = k0 + 1
            fire_idx(k1, 1)
            drain_sg(0)
            fire_ha(0)
            drain_idx(1)
            fire_sg(1)
            drain_ha(0)
            fire_idx(k0 + 2, 0)
            fire_store(k0, 0)
            drain_sg(1)
            fire_ha(1)
            drain_idx(0)
            drain_store(0)
            fire_sg(0)
            drain_ha(1)
            fire_store(k1, 1)
            drain_store(1)
            return carry

        lax.fori_loop(0, (nb - 2) // 2, body, 0)
        fire_idx(nb - 1, 1)
        drain_sg(0)
        fire_ha(0)
        drain_idx(1)
        fire_sg(1)
        drain_ha(0)
        fire_store(nb - 2, 0)
        drain_sg(1)
        fire_ha(1)
        drain_ha(1)
        drain_store(0)
        fire_store(nb - 1, 1)
        if tail:
            assert tail == _CH
            tbase = w * ew + nb * _BLKS
            pltpu.sync_copy(src_hbm.at[pl.ds(tbase, _CH)],
                            ia[0].at[pl.ds(0, _CH)])
            pltpu.sync_copy(dst_hbm.at[pl.ds(tbase, _CH)],
                            ib[0].at[pl.ds(0, _CH)])
            pltpu.async_copy(tbl.at[ia[0].at[pl.ds(0, _CH)]],
                             r0.at[pl.ds(0, _CH)], sem_g).wait()
            pltpu.async_copy(md_hbm.at[ib[0].at[pl.ds(0, _CH)]],
                             r0.at[pl.ds(0, _CH)], sem_h, add=True).wait()
            pltpu.sync_copy(r0.at[pl.ds(0, _CH)],
                            out_hbm.at[pl.ds(tbase, _CH)])
        drain_store(1)

    def scatter_body(e_hbm, dst_hbm, zer_hbm, p_hbm, ix0, ix1, r0, r1, acc,
                     sem_i, sem_r, sem_c):
        cid = lax.axis_index("c")
        sid = lax.axis_index("s")
        w = cid * _NS + sid
        ix = (ix0, ix1)
        rw = (r0, r1)
        pltpu.sync_copy(zer_hbm.at[pl.ds(sid * _SLAB, _SLAB)],
                        acc.at[pl.ds(sid * _SLAB, _SLAB)])
        plsc.subcore_barrier()

        def fire_loads(kb, p):
            base = w * ew + kb * _BLKS
            for j in range(_GBS):
                pltpu.async_copy(dst_hbm.at[pl.ds(base + j * _CH, _CH)],
                                 ix[p][j], sem_i)
            pltpu.async_copy(e_hbm.at[pl.ds(base, _BLKS)], rw[p], sem_r)

        def drain_loads(p):
            for j in range(_GBS):
                pltpu.make_async_copy(dst_hbm.at[pl.ds(0, _CH)], ix[p][j],
                                      sem_i).wait()
            pltpu.make_async_copy(e_hbm.at[pl.ds(0, _BLKS)], rw[p],
                                  sem_r).wait()

        def fire_sadd(p):
            for j in range(_GBS):
                pltpu.async_copy(rw[p].at[pl.ds(j * _CH, _CH)],
                                 acc.at[ix[p][j]], sem_c, add=True)

        def drain_sadd(p):
            for j in range(_GBS):
                pltpu.make_async_copy(rw[p].at[pl.ds(j * _CH, _CH)],
                                      acc.at[ix[p][j]], sem_c).wait()

        fire_loads(0, 0)
        drain_loads(0)

        def body(i, carry):
            k1 = 2 * i + 1
            fire_loads(k1, 1)
            fire_sadd(0)
            drain_loads(1)
            drain_sadd(0)
            fire_loads(k1 + 1, 0)
            fire_sadd(1)
            drain_loads(0)
            drain_sadd(1)
            return carry

        lax.fori_loop(0, (nb - 2) // 2, body, 0)
        fire_loads(nb - 1, 1)
        fire_sadd(0)
        drain_loads(1)
        drain_sadd(0)
        fire_sadd(1)
        drain_sadd(1)
        if tail:
            assert tail == _CH
            tbase = w * ew + nb * _BLKS
            pltpu.sync_copy(dst_hbm.at[pl.ds(tbase, _CH)], ix[0][0])
            pltpu.sync_copy(e_hbm.at[pl.ds(tbase, _CH)],
                            r0.at[pl.ds(0, _CH)])
            pltpu.async_copy(r0.at[pl.ds(0, _CH)], acc.at[ix[0][0]], sem_c,
                             add=True).wait()
        plsc.subcore_barrier()
        pltpu.sync_copy(acc.at[pl.ds(sid * _SLAB, _SLAB)],
                        p_hbm.at[cid, pl.ds(sid * _SLAB, _SLAB)])

    return gather_body, scatter_body


@functools.cache
def _sc_kernels(ne):
    ew = ne // _NW
    nb = ew // _BLKS
    tail = ew - nb * _BLKS
    if nb % 2:
        nb -= 1
        tail += _BLKS
    gather_body, scatter_body = _sc_bodies(ew, nb, tail)
    mesh = plsc.VectorSubcoreMesh(core_axis_name="c", subcore_axis_name="s",
                                  num_cores=_NC, num_subcores=_NS)
    gather = pl.kernel(
        gather_body,
        out_type=jax.ShapeDtypeStruct((ne, HID), jnp.float32),
        mesh=mesh,
        scratch_types=[
            pltpu.VMEM((_BLKS,), jnp.int32),
            pltpu.VMEM((_BLKS,), jnp.int32),
            pltpu.VMEM((_BLKS,), jnp.int32),
            pltpu.VMEM((_BLKS,), jnp.int32),
            pltpu.VMEM((_BLKS, HID), jnp.float32),
            pltpu.VMEM((_BLKS, HID), jnp.float32),
            pltpu.VMEM_SHARED((NPAD, HID), jnp.float32),
            pltpu.SemaphoreType.DMA,
            pltpu.SemaphoreType.DMA,
            pltpu.SemaphoreType.DMA,
            pltpu.SemaphoreType.DMA,
        ],
    )
    scatter = pl.kernel(
        scatter_body,
        out_type=jax.ShapeDtypeStruct((_NC, NPAD, HID), jnp.float32),
        mesh=mesh,
        scratch_types=[
            [pltpu.VMEM((_CH,), jnp.int32) for _ in range(_GBS)],
            [pltpu.VMEM((_CH,), jnp.int32) for _ in range(_GBS)],
            pltpu.VMEM((_BLKS, HID), jnp.float32),
            pltpu.VMEM((_BLKS, HID), jnp.float32),
            pltpu.VMEM_SHARED((NPAD, HID), jnp.float32),
            pltpu.SemaphoreType.DMA,
            pltpu.SemaphoreType.DMA,
            pltpu.SemaphoreType.DMA,
        ],
    )
    return gather, scatter


_EA = 163840
_EB = N_EDGES - _EA


def kernel(node_attr, edge_attr, edge_index, params):
    gather_a, scatter_a = _sc_kernels(_EA)
    gather_b, scatter_b = _sc_kernels(_EB)
    src_a = edge_index[0, :_EA]
    dst_a = edge_index[1, :_EA]
    src_b = edge_index[0, _EA:]
    dst_b = edge_index[1, _EA:]

    na = jnp.pad(node_attr, ((0, NPAD - N_NODES), (0, 5)))
    ea = jnp.pad(edge_attr, ((0, 0), (0, 1)))

    pn = params['node_enc']
    v = _encode(na, jnp.pad(pn['w1'], ((0, 5), (0, 0))), pn['b1'], pn['w2'],
                pn['b2'], pn['g'], pn['beta'], NT)
    pe = params['edge_enc']
    ew1 = jnp.pad(pe['w1'], ((0, 1), (0, 0)))
    e_a = _encode(ea[:_EA], ew1, pe['b1'], pe['w2'], pe['b2'], pe['g'],
                  pe['beta'], ET)
    e_b = _encode(ea[_EA:], ew1, pe['b1'], pe['w2'], pe['b2'], pe['g'],
                  pe['beta'], ET)

    zer = jnp.zeros((NPAD, HID), jnp.float32)
    blocks = params['blocks']
    w10 = blocks[0]['edge']['w1']
    ms, md = _proj(v, w10[HID:2 * HID], w10[2 * HID:])
    for bi, blk in enumerate(blocks):
        nxt = blocks[bi + 1] if bi + 1 < len(blocks) else None
        be_, bn = blk['edge'], blk['node']
        w1 = be_['w1']
        ew = (w1[:HID], be_['b1'], be_['w2'], be_['b2'], be_['g'], be_['beta'])
        g_a = gather_a(ms, md, src_a, dst_a)
        g_b = gather_b(ms, md, src_b, dst_b)
        e_a = _edge_stage(e_a, g_a, *ew)
        p_a = scatter_a(e_a, dst_a, zer)
        e_b = _edge_stage(e_b, g_b, *ew)
        p_b = scatter_b(e_b, dst_b, zer)
        nw1 = bn['w1']
        if nxt is not None:
            nw1e = nxt['edge']['w1']
            v, ms, md = _node_proj_stage(
                v, p_a, p_b, nw1[:HID], nw1[HID:], bn['b1'], bn['w2'],
                bn['b2'], bn['g'], bn['beta'], nw1e[HID:2 * HID],
                nw1e[2 * HID:])
        else:
            v = _node_stage(v, p_a, p_b, nw1[:HID], nw1[HID:], bn['b1'],
                            bn['w2'], bn['b2'], bn['g'], bn['beta'])

    pd = params['decoder']
    w2p = jnp.pad(pd['w2'], ((0, 0), (0, HID - pd['w2'].shape[1])))
    b2p = jnp.pad(pd['b2'], (0, HID - pd['b2'].shape[0]))
    out = _decode(v, pd['w1'], pd['b1'], w2p, b2p)
    return out[:N_NODES, :3]
